# baseline (device time: 117060 ns/iter reference)
import jax
import jax.numpy as jnp
from jax import lax
from jax.experimental import pallas as pl
from jax.experimental.pallas import tpu as pltpu

N_DEV = 4
SQ = 2048
SKV = 2048
HQ = 8
DH = 128
BLK = 64
SCALE = 0.08838834764831843
SCALE2 = SCALE * 1.4426950408889634


def _ring_allreduce(partial):
    S, D = partial.shape
    C = S // N_DEV
    H = 2 * (N_DEV - 1)

    def body(p_ref, out_ref, comm_ref, send_sems, recv_sems):
        my = lax.axis_index("i")
        left = lax.rem(my + N_DEV - 1, N_DEV)
        right = lax.rem(my + 1, N_DEV)

        barrier = pltpu.get_barrier_semaphore()
        for nbr in (left, right):
            pl.semaphore_signal(
                barrier, inc=1,
                device_id=(nbr,), device_id_type=pl.DeviceIdType.MESH,
            )
        pl.semaphore_wait(barrier, 2)

        def local_chunk(c):
            return p_ref[pl.ds(c * C, C), :]

        def store_out(c, h):
            out_ref[pl.ds(c * C, C), :] = comm_ref[h].astype(jnp.float32)

        def hop(h, src):
            rdma = pltpu.make_async_remote_copy(
                src_ref=src,
                dst_ref=comm_ref.at[h],
                send_sem=send_sems.at[h],
                recv_sem=recv_sems.at[h],
                device_id=(right,),
                device_id_type=pl.DeviceIdType.MESH,
            )
            rdma.start()
            rdma.wait()

        comm_ref[H] = local_chunk(my)
        for h in range(N_DEV - 1):
            hop(h, comm_ref.at[H if h == 0 else h - 1])
            c_recv = lax.rem(my - h - 1 + N_DEV, N_DEV)
            comm_ref[h] = comm_ref[h] + local_chunk(c_recv)

        store_out(lax.rem(my + 1, N_DEV), N_DEV - 2)

        for g in range(N_DEV - 1):
            h = (N_DEV - 1) + g
            hop(h, comm_ref.at[h - 1])
            store_out(lax.rem(my - g + N_DEV, N_DEV), h)

    return pl.pallas_call(
        body,
        out_shape=jax.ShapeDtypeStruct((S, D), jnp.float32),
        in_specs=[pl.BlockSpec(memory_space=pltpu.VMEM)],
        out_specs=pl.BlockSpec(memory_space=pltpu.VMEM),
        scratch_shapes=[
            pltpu.VMEM((H + 1, C, D), partial.dtype),
            pltpu.SemaphoreType.DMA((H,)),
            pltpu.SemaphoreType.DMA((H,)),
        ],
        compiler_params=pltpu.CompilerParams(collective_id=0),
    )(partial)


QB = 512
KB = 512
NQ = SQ // QB
DM = 1024


def _attention_partial(x2, Wq, Kh, Vh, Wo):

    def body(x_ref, wq_ref, k_ref, v_ref, wo_ref, out_ref, acc_ref):
        t = pl.program_id(0)
        h = pl.program_id(1)

        q = jnp.dot(x_ref[...], wq_ref[...],
                    preferred_element_type=jnp.float32).astype(jnp.bfloat16)

        ri = lax.broadcasted_iota(jnp.int32, (QB, KB), 0) // BLK
        ci = lax.broadcasted_iota(jnp.int32, (QB, KB), 1) // BLK
        diag_mask = ci <= ri

        def kv_step(j, carry):
            s_sum, acc = carry
            k = k_ref[0, pl.ds(j * KB, KB), :]
            v = v_ref[0, pl.ds(j * KB, KB), :]
            s = lax.dot_general(
                q, k, (((1,), (1,)), ((), ())),
                preferred_element_type=jnp.float32) * SCALE
            p = jnp.exp(s)
            p = jnp.where(j == t, jnp.where(diag_mask, p, 0.0), p)
            s_sum = s_sum + p.sum(axis=1, keepdims=True)
            acc = acc + jnp.dot(p.astype(jnp.bfloat16), v,
                                preferred_element_type=jnp.float32)
            return s_sum, acc

        s_sum, acc = lax.fori_loop(
            0, t + 1, kv_step,
            (jnp.zeros((QB, 1), jnp.float32), jnp.zeros((QB, DH), jnp.float32)),
        )
        ctx = (acc / s_sum).astype(jnp.bfloat16)
        contrib = jnp.dot(ctx, wo_ref[...],
                          preferred_element_type=jnp.float32)

        @pl.when(h == 0)
        def _():
            acc_ref[...] = contrib

        @pl.when(h > 0)
        def _():
            acc_ref[...] = acc_ref[...] + contrib

        @pl.when(h == HQ - 1)
        def _():
            out_ref[...] = acc_ref[...].astype(jnp.bfloat16)

    return pl.pallas_call(
        body,
        grid=(NQ, HQ),
        in_specs=[
            pl.BlockSpec((QB, DM), lambda t, h: (t, 0)),
            pl.BlockSpec((DM, DH), lambda t, h: (0, h)),
            pl.BlockSpec((1, SKV, DH), lambda t, h: (h, 0, 0)),
            pl.BlockSpec((1, SKV, DH), lambda t, h: (h, 0, 0)),
            pl.BlockSpec((DH, DM), lambda t, h: (h, 0)),
        ],
        out_specs=pl.BlockSpec((QB, DM), lambda t, h: (t, 0)),
        out_shape=jax.ShapeDtypeStruct((SQ, DM), jnp.bfloat16),
        scratch_shapes=[pltpu.VMEM((QB, DM), jnp.float32)],
        compiler_params=pltpu.CompilerParams(
            dimension_semantics=("arbitrary", "arbitrary"),
        ),
    )(x2, Wq, Kh, Vh, Wo)


def _fused(x2, Wq, Kh, Vh, Wo):
    C = SQ // N_DEV
    STG = N_DEV - 1

    def body(x_ref, wq_ref, khbm_ref, vhbm_ref, wo_ref, out_ref,
             comm_ref, ag_ref, acc_ref, q_ref, pv_ref, ssum_ref,
             k_ref, v_ref, stage_ref,
             send_sems, recv_sems, ag_send_sems, ag_recv_sems, dma_sems):
        my = lax.axis_index("i")
        left = lax.rem(my + N_DEV - 1, N_DEV)
        right = lax.rem(my + 1, N_DEV)

        base = my * HQ

        def kv_copy(i):
            src = (khbm_ref if i % 2 == 0 else vhbm_ref).at[:, base + i // 2, :]
            return pltpu.make_async_copy(src, stage_ref.at[i], dma_sems.at[i])

        for i in range(2 * HQ):
            kv_copy(i).start()

        barrier = pltpu.get_barrier_semaphore()
        for nbr in (left, right):
            pl.semaphore_signal(
                barrier, inc=1,
                device_id=(nbr,), device_id_type=pl.DeviceIdType.MESH,
            )
        pl.semaphore_wait(barrier, 2)

        ri = lax.broadcasted_iota(jnp.int32, (C, KB), 0) // BLK
        ci = lax.broadcasted_iota(jnp.int32, (C, KB), 1) // BLK
        diag_mask = ci <= ri

        def compute_chunk(c, first=False):
            q_ref[...] = (jnp.dot(
                x_ref[pl.ds(c * C, C), :], wq_ref[...],
                preferred_element_type=jnp.float32,
            ) * SCALE2).astype(jnp.bfloat16)
            pv_ref[...] = jnp.zeros((C, HQ * DH), jnp.float32)
            ssum_ref[...] = jnp.zeros((HQ, C), jnp.float32)

            def attend(h, j, masked):
                cols = pl.ds(h * DH, DH)
                k = k_ref[h, pl.ds(j * KB, KB), :]
                v = v_ref[h, pl.ds(j * KB, KB), :]
                s = lax.dot_general(
                    q_ref[:, cols], k, (((1,), (1,)), ((), ())),
                    preferred_element_type=jnp.float32)
                p = jnp.exp2(s)
                if masked:
                    p = jnp.where(diag_mask, p, 0.0)
                ssum_ref[h, :] = ssum_ref[h, :] + p.sum(axis=1)
                pv_ref[:, cols] = pv_ref[:, cols] + jnp.dot(
                    p.astype(jnp.bfloat16), v,
                    preferred_element_type=jnp.float32)

            if first:
                for h in range(HQ):
                    kv_copy(2 * h).wait()
                    k_ref[h] = stage_ref[2 * h].astype(jnp.bfloat16)
                    kv_copy(2 * h + 1).wait()
                    v_ref[h] = stage_ref[2 * h + 1].astype(jnp.bfloat16)

                    def kv1(j, _, h=h):
                        attend(h, j, masked=False)
                        return 0

                    lax.fori_loop(0, c, kv1, 0)
                    attend(h, c, masked=True)
            else:
                def kv_step(j, _):
                    for h in range(HQ):
                        attend(h, j, masked=False)
                    return 0

                lax.fori_loop(0, c, kv_step, 0)
                for h in range(HQ):
                    attend(h, c, masked=True)

            for h in range(HQ):
                cols = pl.ds(h * DH, DH)
                recip = (1.0 / ssum_ref[h, :])[:, None]
                q_ref[:, cols] = (pv_ref[:, cols] * recip).astype(jnp.bfloat16)
            acc_ref[...] = jnp.dot(q_ref[...], wo_ref[...],
                                   preferred_element_type=jnp.float32)

        def make_hop(h, src_slot, to=None):
            return pltpu.make_async_remote_copy(
                src_ref=comm_ref.at[src_slot],
                dst_ref=comm_ref.at[h],
                send_sem=send_sems.at[h],
                recv_sem=recv_sems.at[h],
                device_id=(right if to is None else to,),
                device_id_type=pl.DeviceIdType.MESH,
            )

        hops = []
        for t in range(N_DEV):
            c = lax.rem(my - t + N_DEV, N_DEV)
            compute_chunk(c, first=(t == 0))
            if t == 0:
                comm_ref[STG] = acc_ref[...].astype(jnp.bfloat16)
                hop = make_hop(0, STG)
                hop.start()
                hops.append(hop)
            else:
                hops[t - 1].wait_recv()
                comm_ref[t - 1] = (
                    comm_ref[t - 1].astype(jnp.float32) + acc_ref[...]
                ).astype(jnp.bfloat16)
                if t < N_DEV - 1:
                    hop = make_hop(t, t - 1)
                    hop.start()
                    hops.append(hop)

        out_ref[pl.ds(lax.rem(my + 1, N_DEV) * C, C), :] = comm_ref[N_DEV - 2]

        Dh2 = DM // 2

        def ag_hop(slot, src, to):
            return pltpu.make_async_remote_copy(
                src_ref=src,
                dst_ref=ag_ref.at[slot],
                send_sem=ag_send_sems.at[slot],
                recv_sem=ag_recv_sems.at[slot],
                device_id=(to,),
                device_id_type=pl.DeviceIdType.MESH,
            )

        red = comm_ref.at[N_DEV - 2]
        a_src = red.at[:, pl.ds(0, Dh2)]
        b_src = red.at[:, pl.ds(Dh2, Dh2)]
        ar1 = ag_hop(0, a_src, right)
        al1 = ag_hop(1, a_src, left)
        bl1 = ag_hop(3, b_src, left)
        br1 = ag_hop(4, b_src, right)
        for hop in (ar1, al1, bl1, br1):
            hop.start()
            hops.append(hop)

        def store(chunk_off, col_off, slot):
            out_ref[pl.ds(lax.rem(my + chunk_off, N_DEV) * C, C),
                    pl.ds(col_off, Dh2)] = ag_ref[slot]

        ar1.wait_recv()
        ar2 = ag_hop(2, ag_ref.at[0], right)
        ar2.start()
        hops.append(ar2)
        store(0, 0, 0)
        bl1.wait_recv()
        bl2 = ag_hop(5, ag_ref.at[3], left)
        bl2.start()
        hops.append(bl2)
        store(2, Dh2, 3)
        al1.wait_recv()
        store(2, 0, 1)
        br1.wait_recv()
        store(0, Dh2, 4)
        ar2.wait_recv()
        store(3, 0, 2)
        bl2.wait_recv()
        store(3, Dh2, 5)

        for hop in hops:
            hop.wait_send()

    return pl.pallas_call(
        body,
        out_shape=jax.ShapeDtypeStruct((SQ, DM), jnp.bfloat16),
        in_specs=[
            pl.BlockSpec(memory_space=pltpu.VMEM),
            pl.BlockSpec(memory_space=pltpu.VMEM),
            pl.BlockSpec(memory_space=pl.ANY),
            pl.BlockSpec(memory_space=pl.ANY),
            pl.BlockSpec(memory_space=pltpu.VMEM),
        ],
        out_specs=pl.BlockSpec(memory_space=pltpu.VMEM),
        scratch_shapes=[
            pltpu.VMEM((N_DEV, C, DM), jnp.bfloat16),
            pltpu.VMEM((6, C, DM // 2), jnp.bfloat16),
            pltpu.VMEM((C, DM), jnp.float32),
            pltpu.VMEM((C, HQ * DH), jnp.bfloat16),
            pltpu.VMEM((C, HQ * DH), jnp.float32),
            pltpu.VMEM((HQ, C), jnp.float32),
            pltpu.VMEM((HQ, SKV, DH), jnp.bfloat16),
            pltpu.VMEM((HQ, SKV, DH), jnp.bfloat16),
            pltpu.VMEM((2 * HQ, SKV, DH), jnp.float32),
            pltpu.SemaphoreType.DMA((N_DEV - 1,)),
            pltpu.SemaphoreType.DMA((N_DEV - 1,)),
            pltpu.SemaphoreType.DMA((6,)),
            pltpu.SemaphoreType.DMA((6,)),
            pltpu.SemaphoreType.DMA((2 * HQ,)),
        ],
        compiler_params=pltpu.CompilerParams(
            collective_id=0,
            vmem_limit_bytes=100 * 1024 * 1024,
        ),
    )(x2, Wq, Kh, Vh, Wo)


def kernel(x, Wq, K_ext, V_ext, Wo):
    my = lax.axis_index("i")

    xb = x[0].astype(jnp.bfloat16)
    out = _fused(
        xb, Wq.astype(jnp.bfloat16), K_ext[0], V_ext[0],
        Wo.astype(jnp.bfloat16)
    )
    return out[None]


# device time: 114848 ns/iter; 1.0193x vs baseline; 1.0193x over previous
import jax
import jax.numpy as jnp
from jax import lax
from jax.experimental import pallas as pl
from jax.experimental.pallas import tpu as pltpu

N_DEV = 4
SQ = 2048
SKV = 2048
HQ = 8
DH = 128
BLK = 64
SCALE = 0.08838834764831843
SCALE2 = SCALE * 1.4426950408889634


def _ring_allreduce(partial):
    S, D = partial.shape
    C = S // N_DEV
    H = 2 * (N_DEV - 1)

    def body(p_ref, out_ref, comm_ref, send_sems, recv_sems):
        my = lax.axis_index("i")
        left = lax.rem(my + N_DEV - 1, N_DEV)
        right = lax.rem(my + 1, N_DEV)

        barrier = pltpu.get_barrier_semaphore()
        for nbr in (left, right):
            pl.semaphore_signal(
                barrier, inc=1,
                device_id=(nbr,), device_id_type=pl.DeviceIdType.MESH,
            )
        pl.semaphore_wait(barrier, 2)

        def local_chunk(c):
            return p_ref[pl.ds(c * C, C), :]

        def store_out(c, h):
            out_ref[pl.ds(c * C, C), :] = comm_ref[h].astype(jnp.float32)

        def hop(h, src):
            rdma = pltpu.make_async_remote_copy(
                src_ref=src,
                dst_ref=comm_ref.at[h],
                send_sem=send_sems.at[h],
                recv_sem=recv_sems.at[h],
                device_id=(right,),
                device_id_type=pl.DeviceIdType.MESH,
            )
            rdma.start()
            rdma.wait()

        comm_ref[H] = local_chunk(my)
        for h in range(N_DEV - 1):
            hop(h, comm_ref.at[H if h == 0 else h - 1])
            c_recv = lax.rem(my - h - 1 + N_DEV, N_DEV)
            comm_ref[h] = comm_ref[h] + local_chunk(c_recv)

        store_out(lax.rem(my + 1, N_DEV), N_DEV - 2)

        for g in range(N_DEV - 1):
            h = (N_DEV - 1) + g
            hop(h, comm_ref.at[h - 1])
            store_out(lax.rem(my - g + N_DEV, N_DEV), h)

    return pl.pallas_call(
        body,
        out_shape=jax.ShapeDtypeStruct((S, D), jnp.float32),
        in_specs=[pl.BlockSpec(memory_space=pltpu.VMEM)],
        out_specs=pl.BlockSpec(memory_space=pltpu.VMEM),
        scratch_shapes=[
            pltpu.VMEM((H + 1, C, D), partial.dtype),
            pltpu.SemaphoreType.DMA((H,)),
            pltpu.SemaphoreType.DMA((H,)),
        ],
        compiler_params=pltpu.CompilerParams(collective_id=0),
    )(partial)


QB = 512
KB = 512
NQ = SQ // QB
DM = 1024


def _attention_partial(x2, Wq, Kh, Vh, Wo):

    def body(x_ref, wq_ref, k_ref, v_ref, wo_ref, out_ref, acc_ref):
        t = pl.program_id(0)
        h = pl.program_id(1)

        q = jnp.dot(x_ref[...], wq_ref[...],
                    preferred_element_type=jnp.float32).astype(jnp.bfloat16)

        ri = lax.broadcasted_iota(jnp.int32, (QB, KB), 0) // BLK
        ci = lax.broadcasted_iota(jnp.int32, (QB, KB), 1) // BLK
        diag_mask = ci <= ri

        def kv_step(j, carry):
            s_sum, acc = carry
            k = k_ref[0, pl.ds(j * KB, KB), :]
            v = v_ref[0, pl.ds(j * KB, KB), :]
            s = lax.dot_general(
                q, k, (((1,), (1,)), ((), ())),
                preferred_element_type=jnp.float32) * SCALE
            p = jnp.exp(s)
            p = jnp.where(j == t, jnp.where(diag_mask, p, 0.0), p)
            s_sum = s_sum + p.sum(axis=1, keepdims=True)
            acc = acc + jnp.dot(p.astype(jnp.bfloat16), v,
                                preferred_element_type=jnp.float32)
            return s_sum, acc

        s_sum, acc = lax.fori_loop(
            0, t + 1, kv_step,
            (jnp.zeros((QB, 1), jnp.float32), jnp.zeros((QB, DH), jnp.float32)),
        )
        ctx = (acc / s_sum).astype(jnp.bfloat16)
        contrib = jnp.dot(ctx, wo_ref[...],
                          preferred_element_type=jnp.float32)

        @pl.when(h == 0)
        def _():
            acc_ref[...] = contrib

        @pl.when(h > 0)
        def _():
            acc_ref[...] = acc_ref[...] + contrib

        @pl.when(h == HQ - 1)
        def _():
            out_ref[...] = acc_ref[...].astype(jnp.bfloat16)

    return pl.pallas_call(
        body,
        grid=(NQ, HQ),
        in_specs=[
            pl.BlockSpec((QB, DM), lambda t, h: (t, 0)),
            pl.BlockSpec((DM, DH), lambda t, h: (0, h)),
            pl.BlockSpec((1, SKV, DH), lambda t, h: (h, 0, 0)),
            pl.BlockSpec((1, SKV, DH), lambda t, h: (h, 0, 0)),
            pl.BlockSpec((DH, DM), lambda t, h: (h, 0)),
        ],
        out_specs=pl.BlockSpec((QB, DM), lambda t, h: (t, 0)),
        out_shape=jax.ShapeDtypeStruct((SQ, DM), jnp.bfloat16),
        scratch_shapes=[pltpu.VMEM((QB, DM), jnp.float32)],
        compiler_params=pltpu.CompilerParams(
            dimension_semantics=("arbitrary", "arbitrary"),
        ),
    )(x2, Wq, Kh, Vh, Wo)


def _fused(x2, Wq, Kh, Vh, Wo):
    C = SQ // N_DEV
    STG = N_DEV - 1

    def body(x_ref, wq_ref, khbm_ref, vhbm_ref, wo_ref, out_ref,
             comm_ref, ag_ref, acc_ref, q_ref, pv_ref, ssum_ref,
             k_ref, v_ref, stage_ref,
             send_sems, recv_sems, ag_send_sems, ag_recv_sems, dma_sems):
        my = lax.axis_index("i")
        left = lax.rem(my + N_DEV - 1, N_DEV)
        right = lax.rem(my + 1, N_DEV)

        base = my * HQ

        def kv_copy(i, slot):
            src = (khbm_ref if i % 2 == 0 else vhbm_ref).at[:, base + i // 2, :]
            return pltpu.make_async_copy(src, stage_ref.at[slot], dma_sems.at[slot])

        n_cp = 2 * HQ
        for i in range(min(4, n_cp)):
            kv_copy(i, i % 4).start()

        barrier = pltpu.get_barrier_semaphore()
        for nbr in (left, right):
            pl.semaphore_signal(
                barrier, inc=1,
                device_id=(nbr,), device_id_type=pl.DeviceIdType.MESH,
            )
        pl.semaphore_wait(barrier, 2)

        for i in range(n_cp):
            slot = i % 4
            kv_copy(i, slot).wait()
            dst = k_ref if i % 2 == 0 else v_ref
            dst[i // 2] = stage_ref[slot].astype(jnp.bfloat16)
            if i + 4 < n_cp:
                kv_copy(i + 4, slot).start()

        ri = lax.broadcasted_iota(jnp.int32, (C, KB), 0) // BLK
        ci = lax.broadcasted_iota(jnp.int32, (C, KB), 1) // BLK
        diag_mask = ci <= ri

        def compute_chunk(c):
            q_ref[...] = (jnp.dot(
                x_ref[pl.ds(c * C, C), :], wq_ref[...],
                preferred_element_type=jnp.float32,
            ) * SCALE2).astype(jnp.bfloat16)
            pv_ref[...] = jnp.zeros((C, HQ * DH), jnp.float32)
            ssum_ref[...] = jnp.zeros((HQ, C), jnp.float32)

            def attend(h, j, masked):
                cols = pl.ds(h * DH, DH)
                k = k_ref[h, pl.ds(j * KB, KB), :]
                v = v_ref[h, pl.ds(j * KB, KB), :]
                s = lax.dot_general(
                    q_ref[:, cols], k, (((1,), (1,)), ((), ())),
                    preferred_element_type=jnp.float32)
                p = jnp.exp2(s)
                if masked:
                    p = jnp.where(diag_mask, p, 0.0)
                ssum_ref[h, :] = ssum_ref[h, :] + p.sum(axis=1)
                pv_ref[:, cols] = pv_ref[:, cols] + jnp.dot(
                    p.astype(jnp.bfloat16), v,
                    preferred_element_type=jnp.float32)

            def kv_step(j, _):
                for h in range(HQ):
                    attend(h, j, masked=False)
                return 0

            lax.fori_loop(0, c, kv_step, 0)
            for h in range(HQ):
                attend(h, c, masked=True)

            for h in range(HQ):
                cols = pl.ds(h * DH, DH)
                recip = (1.0 / ssum_ref[h, :])[:, None]
                q_ref[:, cols] = (pv_ref[:, cols] * recip).astype(jnp.bfloat16)
            acc_ref[...] = jnp.dot(q_ref[...], wo_ref[...],
                                   preferred_element_type=jnp.float32)

        def make_hop(h, src_slot, to=None):
            return pltpu.make_async_remote_copy(
                src_ref=comm_ref.at[src_slot],
                dst_ref=comm_ref.at[h],
                send_sem=send_sems.at[h],
                recv_sem=recv_sems.at[h],
                device_id=(right if to is None else to,),
                device_id_type=pl.DeviceIdType.MESH,
            )

        hops = []
        for t in range(N_DEV):
            c = lax.rem(my - t + N_DEV, N_DEV)
            compute_chunk(c)
            if t == 0:
                comm_ref[STG] = acc_ref[...].astype(jnp.bfloat16)
                hop = make_hop(0, STG)
                hop.start()
                hops.append(hop)
            else:
                hops[t - 1].wait_recv()
                comm_ref[t - 1] = (
                    comm_ref[t - 1].astype(jnp.float32) + acc_ref[...]
                ).astype(jnp.bfloat16)
                if t < N_DEV - 1:
                    hop = make_hop(t, t - 1)
                    hop.start()
                    hops.append(hop)

        out_ref[pl.ds(lax.rem(my + 1, N_DEV) * C, C), :] = comm_ref[N_DEV - 2]

        Dh2 = DM // 2

        def ag_hop(slot, src, to):
            return pltpu.make_async_remote_copy(
                src_ref=src,
                dst_ref=ag_ref.at[slot],
                send_sem=ag_send_sems.at[slot],
                recv_sem=ag_recv_sems.at[slot],
                device_id=(to,),
                device_id_type=pl.DeviceIdType.MESH,
            )

        red = comm_ref.at[N_DEV - 2]
        a_src = red.at[:, pl.ds(0, Dh2)]
        b_src = red.at[:, pl.ds(Dh2, Dh2)]
        ar1 = ag_hop(0, a_src, right)
        al1 = ag_hop(1, a_src, left)
        bl1 = ag_hop(3, b_src, left)
        br1 = ag_hop(4, b_src, right)
        for hop in (ar1, al1, bl1, br1):
            hop.start()
            hops.append(hop)

        def store(chunk_off, col_off, slot):
            out_ref[pl.ds(lax.rem(my + chunk_off, N_DEV) * C, C),
                    pl.ds(col_off, Dh2)] = ag_ref[slot]

        ar1.wait_recv()
        ar2 = ag_hop(2, ag_ref.at[0], right)
        ar2.start()
        hops.append(ar2)
        store(0, 0, 0)
        bl1.wait_recv()
        bl2 = ag_hop(5, ag_ref.at[3], left)
        bl2.start()
        hops.append(bl2)
        store(2, Dh2, 3)
        al1.wait_recv()
        store(2, 0, 1)
        br1.wait_recv()
        store(0, Dh2, 4)
        ar2.wait_recv()
        store(3, 0, 2)
        bl2.wait_recv()
        store(3, Dh2, 5)

        for hop in hops:
            hop.wait_send()

    return pl.pallas_call(
        body,
        out_shape=jax.ShapeDtypeStruct((SQ, DM), jnp.bfloat16),
        in_specs=[
            pl.BlockSpec(memory_space=pltpu.VMEM),
            pl.BlockSpec(memory_space=pltpu.VMEM),
            pl.BlockSpec(memory_space=pl.ANY),
            pl.BlockSpec(memory_space=pl.ANY),
            pl.BlockSpec(memory_space=pltpu.VMEM),
        ],
        out_specs=pl.BlockSpec(memory_space=pltpu.VMEM),
        scratch_shapes=[
            pltpu.VMEM((N_DEV, C, DM), jnp.bfloat16),
            pltpu.VMEM((6, C, DM // 2), jnp.bfloat16),
            pltpu.VMEM((C, DM), jnp.float32),
            pltpu.VMEM((C, HQ * DH), jnp.bfloat16),
            pltpu.VMEM((C, HQ * DH), jnp.float32),
            pltpu.VMEM((HQ, C), jnp.float32),
            pltpu.VMEM((HQ, SKV, DH), jnp.bfloat16),
            pltpu.VMEM((HQ, SKV, DH), jnp.bfloat16),
            pltpu.VMEM((4, SKV, DH), jnp.float32),
            pltpu.SemaphoreType.DMA((N_DEV - 1,)),
            pltpu.SemaphoreType.DMA((N_DEV - 1,)),
            pltpu.SemaphoreType.DMA((6,)),
            pltpu.SemaphoreType.DMA((6,)),
            pltpu.SemaphoreType.DMA((4,)),
        ],
        compiler_params=pltpu.CompilerParams(
            collective_id=0,
            vmem_limit_bytes=100 * 1024 * 1024,
        ),
    )(x2, Wq, Kh, Vh, Wo)


def kernel(x, Wq, K_ext, V_ext, Wo):
    my = lax.axis_index("i")

    xb = x[0].astype(jnp.bfloat16)
    out = _fused(
        xb, Wq.astype(jnp.bfloat16), K_ext[0], V_ext[0],
        Wo.astype(jnp.bfloat16)
    )
    return out[None]


# device time: 103729 ns/iter; 1.1285x vs baseline; 1.1072x over previous
import jax
import jax.numpy as jnp
from jax import lax
from jax.experimental import pallas as pl
from jax.experimental.pallas import tpu as pltpu

N_DEV = 4
SQ = 2048
SKV = 2048
HQ = 8
DH = 128
BLK = 64
SCALE = 0.08838834764831843
SCALE2 = SCALE * 1.4426950408889634


def _ring_allreduce(partial):
    S, D = partial.shape
    C = S // N_DEV
    H = 2 * (N_DEV - 1)

    def body(p_ref, out_ref, comm_ref, send_sems, recv_sems):
        my = lax.axis_index("i")
        left = lax.rem(my + N_DEV - 1, N_DEV)
        right = lax.rem(my + 1, N_DEV)

        barrier = pltpu.get_barrier_semaphore()
        for nbr in (left, right):
            pl.semaphore_signal(
                barrier, inc=1,
                device_id=(nbr,), device_id_type=pl.DeviceIdType.MESH,
            )
        pl.semaphore_wait(barrier, 2)

        def local_chunk(c):
            return p_ref[pl.ds(c * C, C), :]

        def store_out(c, h):
            out_ref[pl.ds(c * C, C), :] = comm_ref[h].astype(jnp.float32)

        def hop(h, src):
            rdma = pltpu.make_async_remote_copy(
                src_ref=src,
                dst_ref=comm_ref.at[h],
                send_sem=send_sems.at[h],
                recv_sem=recv_sems.at[h],
                device_id=(right,),
                device_id_type=pl.DeviceIdType.MESH,
            )
            rdma.start()
            rdma.wait()

        comm_ref[H] = local_chunk(my)
        for h in range(N_DEV - 1):
            hop(h, comm_ref.at[H if h == 0 else h - 1])
            c_recv = lax.rem(my - h - 1 + N_DEV, N_DEV)
            comm_ref[h] = comm_ref[h] + local_chunk(c_recv)

        store_out(lax.rem(my + 1, N_DEV), N_DEV - 2)

        for g in range(N_DEV - 1):
            h = (N_DEV - 1) + g
            hop(h, comm_ref.at[h - 1])
            store_out(lax.rem(my - g + N_DEV, N_DEV), h)

    return pl.pallas_call(
        body,
        out_shape=jax.ShapeDtypeStruct((S, D), jnp.float32),
        in_specs=[pl.BlockSpec(memory_space=pltpu.VMEM)],
        out_specs=pl.BlockSpec(memory_space=pltpu.VMEM),
        scratch_shapes=[
            pltpu.VMEM((H + 1, C, D), partial.dtype),
            pltpu.SemaphoreType.DMA((H,)),
            pltpu.SemaphoreType.DMA((H,)),
        ],
        compiler_params=pltpu.CompilerParams(collective_id=0),
    )(partial)


QB = 512
KB = 512
NQ = SQ // QB
DM = 1024


def _attention_partial(x2, Wq, Kh, Vh, Wo):

    def body(x_ref, wq_ref, k_ref, v_ref, wo_ref, out_ref, acc_ref):
        t = pl.program_id(0)
        h = pl.program_id(1)

        q = jnp.dot(x_ref[...], wq_ref[...],
                    preferred_element_type=jnp.float32).astype(jnp.bfloat16)

        ri = lax.broadcasted_iota(jnp.int32, (QB, KB), 0) // BLK
        ci = lax.broadcasted_iota(jnp.int32, (QB, KB), 1) // BLK
        diag_mask = ci <= ri

        def kv_step(j, carry):
            s_sum, acc = carry
            k = k_ref[0, pl.ds(j * KB, KB), :]
            v = v_ref[0, pl.ds(j * KB, KB), :]
            s = lax.dot_general(
                q, k, (((1,), (1,)), ((), ())),
                preferred_element_type=jnp.float32) * SCALE
            p = jnp.exp(s)
            p = jnp.where(j == t, jnp.where(diag_mask, p, 0.0), p)
            s_sum = s_sum + p.sum(axis=1, keepdims=True)
            acc = acc + jnp.dot(p.astype(jnp.bfloat16), v,
                                preferred_element_type=jnp.float32)
            return s_sum, acc

        s_sum, acc = lax.fori_loop(
            0, t + 1, kv_step,
            (jnp.zeros((QB, 1), jnp.float32), jnp.zeros((QB, DH), jnp.float32)),
        )
        ctx = (acc / s_sum).astype(jnp.bfloat16)
        contrib = jnp.dot(ctx, wo_ref[...],
                          preferred_element_type=jnp.float32)

        @pl.when(h == 0)
        def _():
            acc_ref[...] = contrib

        @pl.when(h > 0)
        def _():
            acc_ref[...] = acc_ref[...] + contrib

        @pl.when(h == HQ - 1)
        def _():
            out_ref[...] = acc_ref[...].astype(jnp.bfloat16)

    return pl.pallas_call(
        body,
        grid=(NQ, HQ),
        in_specs=[
            pl.BlockSpec((QB, DM), lambda t, h: (t, 0)),
            pl.BlockSpec((DM, DH), lambda t, h: (0, h)),
            pl.BlockSpec((1, SKV, DH), lambda t, h: (h, 0, 0)),
            pl.BlockSpec((1, SKV, DH), lambda t, h: (h, 0, 0)),
            pl.BlockSpec((DH, DM), lambda t, h: (h, 0)),
        ],
        out_specs=pl.BlockSpec((QB, DM), lambda t, h: (t, 0)),
        out_shape=jax.ShapeDtypeStruct((SQ, DM), jnp.bfloat16),
        scratch_shapes=[pltpu.VMEM((QB, DM), jnp.float32)],
        compiler_params=pltpu.CompilerParams(
            dimension_semantics=("arbitrary", "arbitrary"),
        ),
    )(x2, Wq, Kh, Vh, Wo)


def _fused(x2, Wq, Kh, Vh, Wo):
    C = SQ // N_DEV
    STG = N_DEV - 1

    def body(x_ref, wq_ref, khbm_ref, vhbm_ref, wo_ref, out_ref,
             comm_ref, ag_ref, acc_ref, q_ref, pv_ref, ssum_ref,
             k_ref, v_ref, stage_ref,
             send_sems, recv_sems, ag_send_sems, ag_recv_sems, dma_sems):
        my = lax.axis_index("i")
        left = lax.rem(my + N_DEV - 1, N_DEV)
        right = lax.rem(my + 1, N_DEV)

        base = my * HQ

        def kv_copy(i, slot):
            src = (khbm_ref if i % 2 == 0 else vhbm_ref).at[:, base + i // 2, :]
            return pltpu.make_async_copy(src, stage_ref.at[slot], dma_sems.at[slot])

        n_cp = 2 * HQ
        for i in range(min(4, n_cp)):
            kv_copy(i, i % 4).start()

        barrier = pltpu.get_barrier_semaphore()
        for nbr in (left, right):
            pl.semaphore_signal(
                barrier, inc=1,
                device_id=(nbr,), device_id_type=pl.DeviceIdType.MESH,
            )
        pl.semaphore_wait(barrier, 2)

        for i in range(n_cp):
            slot = i % 4
            kv_copy(i, slot).wait()
            dst = k_ref if i % 2 == 0 else v_ref
            dst[i // 2] = stage_ref[slot].astype(jnp.bfloat16)
            if i + 4 < n_cp:
                kv_copy(i + 4, slot).start()

        ri = lax.broadcasted_iota(jnp.int32, (C, KB), 0) // BLK
        ci = lax.broadcasted_iota(jnp.int32, (C, KB), 1) // BLK
        diag_mask = ci <= ri

        def compute_chunk(c):
            q_ref[...] = (jnp.dot(
                x_ref[pl.ds(c * C, C), :], wq_ref[...],
                preferred_element_type=jnp.float32,
            ) * SCALE2).astype(jnp.bfloat16)
            pv_ref[...] = jnp.zeros((C, HQ * DH), jnp.float32)
            ssum_ref[...] = jnp.zeros((HQ, C), jnp.float32)

            def attend(h, j, masked):
                cols = pl.ds(h * DH, DH)
                k = k_ref[h, pl.ds(j * KB, KB), :]
                v = v_ref[h, pl.ds(j * KB, KB), :]
                s = lax.dot_general(
                    q_ref[:, cols], k, (((1,), (1,)), ((), ())),
                    preferred_element_type=jnp.float32)
                p = jnp.exp2(s)
                if masked:
                    p = jnp.where(diag_mask, p, 0.0)
                ssum_ref[h, :] = ssum_ref[h, :] + p.sum(axis=1)
                pv_ref[:, cols] = pv_ref[:, cols] + jnp.dot(
                    p.astype(jnp.bfloat16), v,
                    preferred_element_type=jnp.float32)

            def kv_step(j, _):
                for h in range(HQ):
                    attend(h, j, masked=False)
                return 0

            lax.fori_loop(0, c, kv_step, 0)
            for h in range(HQ):
                attend(h, c, masked=True)

            for h in range(HQ):
                cols = pl.ds(h * DH, DH)
                recip = (1.0 / ssum_ref[h, :])[:, None]
                q_ref[:, cols] = (pv_ref[:, cols] * recip).astype(jnp.bfloat16)
            acc_ref[...] = jnp.dot(q_ref[...], wo_ref[...],
                                   preferred_element_type=jnp.float32)

        def make_hop(h, src_slot, to=None):
            return pltpu.make_async_remote_copy(
                src_ref=comm_ref.at[src_slot],
                dst_ref=comm_ref.at[h],
                send_sem=send_sems.at[h],
                recv_sem=recv_sems.at[h],
                device_id=(right if to is None else to,),
                device_id_type=pl.DeviceIdType.MESH,
            )

        hops = []
        for t in range(N_DEV):
            c = lax.rem(my - t + N_DEV, N_DEV)
            compute_chunk(c)
            if t == 0:
                comm_ref[STG] = acc_ref[...].astype(jnp.bfloat16)
                hop = make_hop(0, STG)
                hop.start()
                hops.append(hop)
            else:
                hops[t - 1].wait_recv()
                comm_ref[t - 1] = (
                    comm_ref[t - 1].astype(jnp.float32) + acc_ref[...]
                ).astype(jnp.bfloat16)
                if t < N_DEV - 1:
                    hop = make_hop(t, t - 1)
                    hop.start()
                    hops.append(hop)

        out_ref[pl.ds(lax.rem(my + 1, N_DEV) * C, C), :] = comm_ref[N_DEV - 2]

        Dh2 = DM // 2

        def ag_hop(slot, src, to):
            return pltpu.make_async_remote_copy(
                src_ref=src,
                dst_ref=ag_ref.at[slot],
                send_sem=ag_send_sems.at[slot],
                recv_sem=ag_recv_sems.at[slot],
                device_id=(to,),
                device_id_type=pl.DeviceIdType.MESH,
            )

        red = comm_ref.at[N_DEV - 2]
        a_src = red.at[:, pl.ds(0, Dh2)]
        b_src = red.at[:, pl.ds(Dh2, Dh2)]
        ar1 = ag_hop(0, a_src, right)
        al1 = ag_hop(1, a_src, left)
        bl1 = ag_hop(3, b_src, left)
        br1 = ag_hop(4, b_src, right)
        for hop in (ar1, al1, bl1, br1):
            hop.start()
            hops.append(hop)

        def store(chunk_off, col_off, slot):
            out_ref[pl.ds(lax.rem(my + chunk_off, N_DEV) * C, C),
                    pl.ds(col_off, Dh2)] = ag_ref[slot]

        ar1.wait_recv()
        ar2 = ag_hop(2, ag_ref.at[0], right)
        ar2.start()
        hops.append(ar2)
        store(0, 0, 0)
        bl1.wait_recv()
        bl2 = ag_hop(5, ag_ref.at[3], left)
        bl2.start()
        hops.append(bl2)
        store(2, Dh2, 3)
        al1.wait_recv()
        store(2, 0, 1)
        br1.wait_recv()
        store(0, Dh2, 4)
        ar2.wait_recv()
        store(3, 0, 2)
        bl2.wait_recv()
        store(3, Dh2, 5)

        for hop in hops:
            hop.wait_send()

    return pl.pallas_call(
        body,
        out_shape=jax.ShapeDtypeStruct((SQ, DM), jnp.bfloat16),
        in_specs=[
            pl.BlockSpec(memory_space=pltpu.VMEM),
            pl.BlockSpec(memory_space=pltpu.VMEM),
            pl.BlockSpec(memory_space=pl.ANY),
            pl.BlockSpec(memory_space=pl.ANY),
            pl.BlockSpec(memory_space=pltpu.VMEM),
        ],
        out_specs=pl.BlockSpec(memory_space=pltpu.VMEM),
        scratch_shapes=[
            pltpu.VMEM((N_DEV, C, DM), jnp.bfloat16),
            pltpu.VMEM((6, C, DM // 2), jnp.bfloat16),
            pltpu.VMEM((C, DM), jnp.float32),
            pltpu.VMEM((C, HQ * DH), jnp.bfloat16),
            pltpu.VMEM((C, HQ * DH), jnp.float32),
            pltpu.VMEM((HQ, C), jnp.float32),
            pltpu.VMEM((HQ, SKV, DH), jnp.bfloat16),
            pltpu.VMEM((HQ, SKV, DH), jnp.bfloat16),
            pltpu.VMEM((4, SKV, DH), jnp.float32),
            pltpu.SemaphoreType.DMA((N_DEV - 1,)),
            pltpu.SemaphoreType.DMA((N_DEV - 1,)),
            pltpu.SemaphoreType.DMA((6,)),
            pltpu.SemaphoreType.DMA((6,)),
            pltpu.SemaphoreType.DMA((4,)),
        ],
        compiler_params=pltpu.CompilerParams(collective_id=0),
    )(x2, Wq, Kh, Vh, Wo)


def kernel(x, Wq, K_ext, V_ext, Wo):
    my = lax.axis_index("i")

    xb = x[0].astype(jnp.bfloat16)
    out = _fused(
        xb, Wq.astype(jnp.bfloat16), K_ext[0], V_ext[0],
        Wo.astype(jnp.bfloat16)
    )
    return out[None]


# device time: 103495 ns/iter; 1.1311x vs baseline; 1.0023x over previous
import jax
import jax.numpy as jnp
from jax import lax
from jax.experimental import pallas as pl
from jax.experimental.pallas import tpu as pltpu

N_DEV = 4
SQ = 2048
SKV = 2048
HQ = 8
DH = 128
DM = 1024
KB = 512
BLK = 64
SCALE = 0.08838834764831843
SCALE2 = SCALE * 1.4426950408889634


def _fused(x2, Wq, Kh, Vh, Wo):
    C = SQ // N_DEV
    STG = N_DEV - 1

    def body(x_ref, wq_ref, khbm_ref, vhbm_ref, wo_ref, out_ref,
             comm_ref, ag_ref, acc_ref, q_ref, pv_ref, ssum_ref,
             k_ref, v_ref, stage_ref,
             send_sems, recv_sems, ag_send_sems, ag_recv_sems, dma_sems):
        my = lax.axis_index("i")
        left = lax.rem(my + N_DEV - 1, N_DEV)
        right = lax.rem(my + 1, N_DEV)

        base = my * HQ

        def kv_copy(i, slot):
            src = (khbm_ref if i % 2 == 0 else vhbm_ref).at[:, base + i // 2, :]
            return pltpu.make_async_copy(src, stage_ref.at[slot], dma_sems.at[slot])

        n_cp = 2 * HQ
        for i in range(min(4, n_cp)):
            kv_copy(i, i % 4).start()

        barrier = pltpu.get_barrier_semaphore()
        for nbr in (left, right):
            pl.semaphore_signal(
                barrier, inc=1,
                device_id=(nbr,), device_id_type=pl.DeviceIdType.MESH,
            )
        pl.semaphore_wait(barrier, 2)

        for i in range(n_cp):
            slot = i % 4
            kv_copy(i, slot).wait()
            dst = k_ref if i % 2 == 0 else v_ref
            dst[i // 2] = stage_ref[slot].astype(jnp.bfloat16)
            if i + 4 < n_cp:
                kv_copy(i + 4, slot).start()

        ri = lax.broadcasted_iota(jnp.int32, (C, KB), 0) // BLK
        ci = lax.broadcasted_iota(jnp.int32, (C, KB), 1) // BLK
        diag_mask = ci <= ri

        def compute_chunk(c):
            q_ref[...] = (jnp.dot(
                x_ref[pl.ds(c * C, C), :], wq_ref[...],
                preferred_element_type=jnp.float32,
            ) * SCALE2).astype(jnp.bfloat16)
            pv_ref[...] = jnp.zeros((C, HQ * DH), jnp.float32)
            ssum_ref[...] = jnp.zeros((HQ, C), jnp.float32)

            def attend(h, j, masked):
                cols = pl.ds(h * DH, DH)
                k = k_ref[h, pl.ds(j * KB, KB), :]
                v = v_ref[h, pl.ds(j * KB, KB), :]
                s = lax.dot_general(
                    q_ref[:, cols], k, (((1,), (1,)), ((), ())),
                    preferred_element_type=jnp.float32)
                p = jnp.exp2(s)
                if masked:
                    p = jnp.where(diag_mask, p, 0.0)
                ssum_ref[h, :] = ssum_ref[h, :] + p.sum(axis=1)
                pv_ref[:, cols] = pv_ref[:, cols] + jnp.dot(
                    p.astype(jnp.bfloat16), v,
                    preferred_element_type=jnp.float32)

            def kv_step(j, _):
                for h in range(HQ):
                    attend(h, j, masked=False)
                return 0

            lax.fori_loop(0, c, kv_step, 0)
            for h in range(HQ):
                attend(h, c, masked=True)

            for h in range(HQ):
                cols = pl.ds(h * DH, DH)
                recip = (1.0 / ssum_ref[h, :])[:, None]
                q_ref[:, cols] = (pv_ref[:, cols] * recip).astype(jnp.bfloat16)
            acc_ref[...] = jnp.dot(q_ref[...], wo_ref[...],
                                   preferred_element_type=jnp.float32)

        def make_hop(h, src_slot, to=None):
            return pltpu.make_async_remote_copy(
                src_ref=comm_ref.at[src_slot],
                dst_ref=comm_ref.at[h],
                send_sem=send_sems.at[h],
                recv_sem=recv_sems.at[h],
                device_id=(right if to is None else to,),
                device_id_type=pl.DeviceIdType.MESH,
            )

        hops = []
        for t in range(N_DEV):
            c = lax.rem(my - t + N_DEV, N_DEV)
            compute_chunk(c)
            if t == 0:
                comm_ref[STG] = acc_ref[...].astype(jnp.bfloat16)
                hop = make_hop(0, STG)
                hop.start()
                hops.append(hop)
            else:
                hops[t - 1].wait_recv()
                comm_ref[t - 1] = (
                    comm_ref[t - 1].astype(jnp.float32) + acc_ref[...]
                ).astype(jnp.bfloat16)
                if t < N_DEV - 1:
                    hop = make_hop(t, t - 1)
                    hop.start()
                    hops.append(hop)

        out_ref[pl.ds(lax.rem(my + 1, N_DEV) * C, C), :] = comm_ref[N_DEV - 2]

        Dh2 = DM // 2

        def ag_hop(slot, src, to):
            return pltpu.make_async_remote_copy(
                src_ref=src,
                dst_ref=ag_ref.at[slot],
                send_sem=ag_send_sems.at[slot],
                recv_sem=ag_recv_sems.at[slot],
                device_id=(to,),
                device_id_type=pl.DeviceIdType.MESH,
            )

        red = comm_ref.at[N_DEV - 2]
        a_src = red.at[:, pl.ds(0, Dh2)]
        b_src = red.at[:, pl.ds(Dh2, Dh2)]
        ar1 = ag_hop(0, a_src, right)
        al1 = ag_hop(1, a_src, left)
        bl1 = ag_hop(3, b_src, left)
        br1 = ag_hop(4, b_src, right)
        for hop in (ar1, al1, bl1, br1):
            hop.start()
            hops.append(hop)

        def store(chunk_off, col_off, slot):
            out_ref[pl.ds(lax.rem(my + chunk_off, N_DEV) * C, C),
                    pl.ds(col_off, Dh2)] = ag_ref[slot]

        ar1.wait_recv()
        ar2 = ag_hop(2, ag_ref.at[0], right)
        ar2.start()
        hops.append(ar2)
        store(0, 0, 0)
        bl1.wait_recv()
        bl2 = ag_hop(5, ag_ref.at[3], left)
        bl2.start()
        hops.append(bl2)
        store(2, Dh2, 3)
        al1.wait_recv()
        store(2, 0, 1)
        br1.wait_recv()
        store(0, Dh2, 4)
        ar2.wait_recv()
        store(3, 0, 2)
        bl2.wait_recv()
        store(3, Dh2, 5)

        for hop in hops:
            hop.wait_send()

    return pl.pallas_call(
        body,
        out_shape=jax.ShapeDtypeStruct((SQ, DM), jnp.bfloat16),
        in_specs=[
            pl.BlockSpec(memory_space=pltpu.VMEM),
            pl.BlockSpec(memory_space=pltpu.VMEM),
            pl.BlockSpec(memory_space=pl.ANY),
            pl.BlockSpec(memory_space=pl.ANY),
            pl.BlockSpec(memory_space=pltpu.VMEM),
        ],
        out_specs=pl.BlockSpec(memory_space=pltpu.VMEM),
        scratch_shapes=[
            pltpu.VMEM((N_DEV, C, DM), jnp.bfloat16),
            pltpu.VMEM((6, C, DM // 2), jnp.bfloat16),
            pltpu.VMEM((C, DM), jnp.float32),
            pltpu.VMEM((C, HQ * DH), jnp.bfloat16),
            pltpu.VMEM((C, HQ * DH), jnp.float32),
            pltpu.VMEM((HQ, C), jnp.float32),
            pltpu.VMEM((HQ, SKV, DH), jnp.bfloat16),
            pltpu.VMEM((HQ, SKV, DH), jnp.bfloat16),
            pltpu.VMEM((4, SKV, DH), jnp.float32),
            pltpu.SemaphoreType.DMA((N_DEV - 1,)),
            pltpu.SemaphoreType.DMA((N_DEV - 1,)),
            pltpu.SemaphoreType.DMA((6,)),
            pltpu.SemaphoreType.DMA((6,)),
            pltpu.SemaphoreType.DMA((4,)),
        ],
        compiler_params=pltpu.CompilerParams(collective_id=0),
    )(x2, Wq, Kh, Vh, Wo)


def kernel(x, Wq, K_ext, V_ext, Wo):
    xb = x[0].astype(jnp.bfloat16)
    out = _fused(
        xb, Wq.astype(jnp.bfloat16), K_ext[0], V_ext[0],
        Wo.astype(jnp.bfloat16)
    )
    return out[None]
